# baseline (speedup 1.0000x reference)
import functools

import jax
import jax.numpy as jnp
from jax import lax
from jax.experimental import pallas as pl
from jax.experimental.pallas import tpu as pltpu
from jax.experimental.pallas import tpu_sc as plsc

B = 50
N = 200
NT = B * N
E = 160000
H = 128
A = 50
L = 4

NTP = 10240
NW = 32
NCH = 40
CH = 128
EP = NW * NCH * CH
PAD_ROW = NT

NCH0 = 16
NCH1 = 64
NCHMX = max(NCH0, NCH1)
EPM_ROWS = 16 * NCH0 + 15 * NCH1 + NCHMX

APW = 80
AP = NW * APW

@functools.cache
def _mesh():
    return plsc.VectorSubcoreMesh(core_axis_name="c", subcore_axis_name="s",
                                  num_cores=2, num_subcores=16)


def _sc_deg_body(dstm_hbm, zeros_hbm, ones_hbm, out_hbm, didx, ones,
                 accum):
    c = lax.axis_index("c")
    s = lax.axis_index("s")
    wid = c * 16 + s
    rps = NTP // 16

    pltpu.sync_copy(zeros_hbm.at[pl.ds(s * rps, rps)],
                    accum.at[pl.ds(s * rps, rps)])
    pltpu.sync_copy(ones_hbm, ones)
    plsc.subcore_barrier()
    pltpu.sync_copy(dstm_hbm.at[pl.ds(wid * NCH, NCH)], didx)

    def _count(j, _):
        pltpu.sync_copy(ones, accum.at[didx.at[j]], add=True)
        return _

    lax.fori_loop(0, NCH, _count, None)
    plsc.subcore_barrier()
    pltpu.sync_copy(accum.at[pl.ds(s * rps, rps)],
                    out_hbm.at[c, pl.ds(s * rps, rps)])


@functools.cache
def _sc_deg():
    return pl.kernel(
        _sc_deg_body,
        out_type=jax.ShapeDtypeStruct((2, NTP, H), jnp.float32),
        mesh=_mesh(),
        scratch_types=[
            pltpu.VMEM((NCH, CH), jnp.int32),
            pltpu.VMEM((CH, H), jnp.float32),
            pltpu.VMEM_SHARED((NTP, H), jnp.float32),
        ],
    )


def _sc_agg_body(h_hbm, srcm_hbm, dstm_hbm, zeros_hbm, out_hbm, sidx, didx,
                 rows0, rows1, accum, g0, g1, s0, s1):
    c = lax.axis_index("c")
    s = lax.axis_index("s")
    wid = c * 16 + s
    rps = NTP // 16

    pltpu.sync_copy(zeros_hbm.at[pl.ds(s * rps, rps)], accum.at[pl.ds(s * rps, rps)])
    plsc.subcore_barrier()

    base = jnp.where(c == 0, s * NCH0, 16 * NCH0 + s * NCH1)
    pltpu.sync_copy(srcm_hbm.at[pl.ds(base, NCHMX)], sidx)
    pltpu.sync_copy(dstm_hbm.at[pl.ds(base, NCHMX)], didx)

    def _run(n):
        pltpu.async_copy(h_hbm.at[sidx.at[0]], rows0, g0)

        def _pair(i, _):
            a = i * 2
            b = a + 1

            @pl.when(i > 0)
            def _():
                pltpu.make_async_copy(rows1, accum.at[didx.at[a]], s1).wait()

            pltpu.async_copy(h_hbm.at[sidx.at[b]], rows1, g1)
            pltpu.make_async_copy(h_hbm.at[sidx.at[a]], rows0, g0).wait()
            pltpu.async_copy(rows0, accum.at[didx.at[a]], s0, add=True)
            pltpu.make_async_copy(h_hbm.at[sidx.at[b]], rows1, g1).wait()
            pltpu.make_async_copy(rows0, accum.at[didx.at[a]], s0).wait()

            @pl.when(i < n // 2 - 1)
            def _():
                pltpu.async_copy(h_hbm.at[sidx.at[a + 2]], rows0, g0)

            pltpu.async_copy(rows1, accum.at[didx.at[b]], s1, add=True)
            return _

        lax.fori_loop(0, n // 2, _pair, None)
        pltpu.make_async_copy(rows1, accum.at[didx.at[0]], s1).wait()

    @pl.when(c == 0)
    def _():
        _run(NCH0)

    @pl.when(c == 1)
    def _():
        _run(NCH1)

    plsc.subcore_barrier()
    pltpu.sync_copy(accum.at[pl.ds(s * rps, rps)],
                    out_hbm.at[c, pl.ds(s * rps, rps)])


@functools.cache
def _sc_agg():
    return pl.kernel(
        _sc_agg_body,
        out_type=jax.ShapeDtypeStruct((2, NTP, H), jnp.float32),
        mesh=_mesh(),
        scratch_types=[
            pltpu.VMEM((NCHMX, CH), jnp.int32),
            pltpu.VMEM((NCHMX, CH), jnp.int32),
            pltpu.VMEM((CH, H), jnp.float32),
            pltpu.VMEM((CH, H), jnp.float32),
            pltpu.VMEM_SHARED((NTP, H), jnp.float32),
            pltpu.SemaphoreType.DMA,
            pltpu.SemaphoreType.DMA,
            pltpu.SemaphoreType.DMA,
            pltpu.SemaphoreType.DMA,
        ],
    )


def _sc_tail_gather_body(z_hbm, gum_hbm, ridm_hbm, cidm_hbm, gidm_hbm,
                         zr_hbm, zc_hbm, gg_hbm, ridx, cidx, gidx, rrows,
                         crows, grows, sem):
    c = lax.axis_index("c")
    s = lax.axis_index("s")
    wid = c * 16 + s
    pltpu.sync_copy(ridm_hbm, ridx)
    pltpu.sync_copy(cidm_hbm, cidx)
    pltpu.sync_copy(gidm_hbm, gidx)
    pltpu.async_copy(z_hbm.at[ridx.at[wid]], rrows, sem).wait()
    pltpu.sync_copy(rrows, zr_hbm.at[pl.ds(wid * APW, APW)])
    pltpu.async_copy(z_hbm.at[cidx.at[wid]], crows, sem).wait()
    pltpu.sync_copy(crows, zc_hbm.at[pl.ds(wid * APW, APW)])
    pltpu.async_copy(gum_hbm.at[gidx.at[wid]], grows, sem).wait()
    pltpu.sync_copy(grows, gg_hbm.at[pl.ds(wid * APW, APW)])


@functools.cache
def _sc_tail_gather():
    return pl.kernel(
        _sc_tail_gather_body,
        out_type=(
            jax.ShapeDtypeStruct((AP, H), jnp.float32),
            jax.ShapeDtypeStruct((AP, H), jnp.float32),
            jax.ShapeDtypeStruct((AP, H), jnp.float32),
        ),
        mesh=_mesh(),
        scratch_types=[
            pltpu.VMEM((NW, APW), jnp.int32),
            pltpu.VMEM((NW, APW), jnp.int32),
            pltpu.VMEM((NW, APW), jnp.int32),
            pltpu.VMEM((APW, H), jnp.float32),
            pltpu.VMEM((APW, H), jnp.float32),
            pltpu.VMEM((APW, H), jnp.float32),
            pltpu.SemaphoreType.DMA,
        ],
    )


def _recip_precise(v):
    r = jax.lax.reciprocal(v)
    r = r * (2.0 - v * r)
    return r * (2.0 - v * r)


def _rsqrt_precise(v):
    r = jax.lax.rsqrt(v)
    r = r * (1.5 - 0.5 * v * r * r)
    return r * (1.5 - 0.5 * v * r * r)


def _prep_body(d_ref, out_ref):
    ssum = d_ref[0] + d_ref[1]
    out_ref[...] = _recip_precise(jnp.maximum(ssum, 1.0))


def _tc_prep(degp):
    return pl.pallas_call(
        _prep_body,
        out_shape=jax.ShapeDtypeStruct((NTP, H), jnp.float32),
    )(degp)


def _layer_body(h_ref, aggp_ref, invd_ref, w1_ref, b1_ref, w2_ref, b2_ref,
                ga_ref, be_ref, out_ref):
    agg = (aggp_ref[0, :NT, :] + aggp_ref[1, :NT, :]) * invd_ref[...]
    x = h_ref[:NT, :] + agg
    z = jnp.dot(x, w1_ref[...], preferred_element_type=jnp.float32) + b1_ref[...]
    z = jnp.maximum(z, 0.0)
    z = jnp.dot(z, w2_ref[...], preferred_element_type=jnp.float32) + b2_ref[...]
    z = jnp.maximum(z, 0.0)
    mu = jnp.mean(z, axis=0, keepdims=True)
    var = jnp.mean((z - mu) ** 2, axis=0, keepdims=True)
    scale = _rsqrt_precise(var + 1e-5) * ga_ref[...]
    out_ref[:NT, :] = (z - mu) * scale + be_ref[...]
    out_ref[NT:, :] = jnp.zeros((NTP - NT, H), jnp.float32)


def _tc_layer(h_pad, aggp, inv_deg, w1, b1, w2, b2, ga, be):
    return pl.pallas_call(
        _layer_body,
        out_shape=jax.ShapeDtypeStruct((NTP, H), jnp.float32),
    )(h_pad, aggp, inv_deg, w1, b1, w2, b2, ga, be)


_BLK = 1000
_GPB = _BLK // N


def _policy_body(h1_ref, h2_ref, h3_ref, h4_ref, w1t_ref, w1b_ref, b1_ref,
                 w2_ref, b2_ref, pw1_ref, pb1_ref, pw2_ref, pb2_ref, out_ref):
    npool = h1_ref[...] + h2_ref[...] + h3_ref[...] + h4_ref[...]
    g_of = jax.lax.broadcasted_iota(jnp.int32, (_GPB, _BLK), 0)
    r_of = jax.lax.broadcasted_iota(jnp.int32, (_GPB, _BLK), 1) // N
    pm = jnp.where(g_of == r_of, 1.0 / N, 0.0)
    gp = jnp.dot(pm, npool, preferred_element_type=jnp.float32)
    q_r = jax.lax.broadcasted_iota(jnp.int32, (_BLK, _GPB), 0) // N
    q_g = jax.lax.broadcasted_iota(jnp.int32, (_BLK, _GPB), 1)
    qm = jnp.where(q_r == q_g, 1.0, 0.0)
    t = (jnp.dot(npool, w1t_ref[...], preferred_element_type=jnp.float32)
         + jnp.dot(qm, jnp.dot(gp, w1b_ref[...],
                               preferred_element_type=jnp.float32),
                   preferred_element_type=jnp.float32)
         + b1_ref[...])
    z = jnp.dot(jnp.tanh(t), w2_ref[...],
                preferred_element_type=jnp.float32) + b2_ref[...]
    for i in range(2):
        t = jnp.dot(z, pw1_ref[i], preferred_element_type=jnp.float32) + pb1_ref[i:i + 1, :]
        z = jnp.dot(jnp.tanh(t), pw2_ref[i],
                    preferred_element_type=jnp.float32) + pb2_ref[i:i + 1, :]
    out_ref[...] = z


def _tc_policy(h1, h2, h3, h4, p0_W1, p0_b1, p0_W2, p0_b2, pW1, pb1, pW2, pb2):
    blk = pl.BlockSpec((_BLK, H), lambda i: (i, 0))
    full = lambda *shape: pl.BlockSpec(shape, lambda i: tuple(0 for _ in shape))
    return pl.pallas_call(
        _policy_body,
        grid=(NT // _BLK,),
        in_specs=[blk, blk, blk, blk,
                  full(H, H), full(H, H), full(1, H),
                  full(H, H), full(1, H),
                  full(2, H, H), full(2, H), full(2, H, H), full(2, H)],
        out_specs=blk,
        out_shape=jax.ShapeDtypeStruct((NT, H), jnp.float32),
    )(h1, h2, h3, h4, p0_W1[:H], p0_W1[H:], p0_b1.reshape(1, H),
      p0_W2, p0_b2.reshape(1, H), pW1, pb1, pW2, pb2)


def _tail_body(zr_ref, zc_ref, gg_ref, sub_ref, pm_ref, out_s_ref, out_lp_ref):
    s = jnp.sum(zr_ref[...] * zc_ref[...], axis=-1)
    k16 = jax.lax.broadcasted_iota(jnp.int32, (B, A, H), 2)
    g = jnp.sum(jnp.where(k16 == sub_ref[...][:, :, None], gg_ref[...], 0.0),
                axis=-1)
    pm = pm_ref[...]
    i_idx = jax.lax.broadcasted_iota(jnp.int32, (B, A, A), 1)
    j_idx = jax.lax.broadcasted_iota(jnp.int32, (B, A, A), 2)
    dup = jnp.any((pm[:, :, None] == pm[:, None, :]) & (i_idx > j_idx), axis=2)
    tot = s + g
    m = jnp.max(tot, axis=1, keepdims=True)
    ia = jax.lax.broadcasted_iota(jnp.int32, (B, A), 1)
    win = jnp.min(jnp.where(tot == m, ia, A), axis=1, keepdims=True)
    sel = ia == win
    aid = jnp.sum(jnp.where(sel, pm, 0), axis=1, keepdims=True)
    s_win = jnp.sum(jnp.where(sel, s, 0.0), axis=1, keepdims=True)
    s_mask = jnp.where(dup, -jnp.inf, s)
    mm = jnp.max(s_mask, axis=1, keepdims=True)
    lse = mm + jnp.log(jnp.sum(jnp.where(dup, 0.0, jnp.exp(s - mm)),
                               axis=1, keepdims=True))
    out_lp_ref[...] = s_win - lse
    out_s_ref[:, 0:1] = aid // N
    out_s_ref[:, 1:2] = aid % N


def _tc_tail(zr, zc, gg, sub, pm):
    return pl.pallas_call(
        _tail_body,
        out_shape=(jax.ShapeDtypeStruct((B, 2), jnp.int32),
                   jax.ShapeDtypeStruct((B, 1), jnp.float32)),
    )(zr, zc, gg, sub, pm)


def kernel(x, edge_index, batch, feasible_actions, gin_W1, gin_b1, gin_W2,
           gin_b2, bn_gamma, bn_beta, p0_W1, p0_b1, p0_W2, p0_b2, pW1, pb1,
           pW2, pb2):
    src = edge_index[0]
    dst = edge_index[1]
    srcm = jnp.full((EPM_ROWS * CH,), PAD_ROW, jnp.int32).at[:E].set(src).reshape(EPM_ROWS, CH)
    dstm = jnp.full((EPM_ROWS * CH,), PAD_ROW, jnp.int32).at[:E].set(dst).reshape(EPM_ROWS, CH)
    zeros_pad = jnp.zeros((NTP, H), jnp.float32)

    degp = _sc_deg()(dstm, zeros_pad, jnp.ones((CH, H), jnp.float32))
    inv_deg = _tc_prep(degp)[:NT, :1]

    h = jnp.zeros((NTP, H), jnp.float32).at[:NT].set(x)
    hs = []
    for l in range(L):
        aggp = _sc_agg()(h, srcm, dstm, zeros_pad)
        h = _tc_layer(h, aggp, inv_deg, gin_W1[l], gin_b1[l].reshape(1, H),
                      gin_W2[l], gin_b2[l].reshape(1, H),
                      bn_gamma[l].reshape(1, H), bn_beta[l].reshape(1, H))
        hs.append(h)

    z = _tc_policy(hs[0], hs[1], hs[2], hs[3], p0_W1, p0_b1, p0_W2, p0_b2,
                   pW1, pb1, pW2, pb2)

    r = feasible_actions[:, :, 0]
    c = feasible_actions[:, :, 1]
    pmat = r * N + c
    boff = jnp.arange(B, dtype=jnp.int32)[:, None]
    ridx = jnp.zeros((AP,), jnp.int32).at[:B * A].set((r + boff * N).ravel())
    cidx = jnp.zeros((AP,), jnp.int32).at[:B * A].set((c + boff * N).ravel())
    gpos = (pmat + boff * (N * N)).ravel()
    gidx = jnp.zeros((AP,), jnp.int32).at[:B * A].set(gpos // H)
    sub = (gpos % H).astype(jnp.int32).reshape(B, A)

    gum = jax.random.gumbel(jax.random.key(42), (B, 1, N * N), jnp.float32)
    gum16 = gum.reshape(B * N * N // H, H)

    zr, zc, gg = _sc_tail_gather()(z, gum16, ridx.reshape(NW, APW),
                                   cidx.reshape(NW, APW), gidx.reshape(NW, APW))
    sampled2, lp = _tc_tail(zr[:B * A].reshape(B, A, H),
                            zc[:B * A].reshape(B, A, H),
                            gg[:B * A].reshape(B, A, H), sub, pmat)
    return (sampled2.reshape(B, 1, 2), lp)


# speedup vs baseline: 1.1115x; 1.1115x over previous
import functools

import jax
import jax.numpy as jnp
from jax import lax
from jax.experimental import pallas as pl
from jax.experimental.pallas import tpu as pltpu
from jax.experimental.pallas import tpu_sc as plsc

B = 50
N = 200
NT = B * N
E = 160000
H = 128
A = 50
L = 4

NTP = 10240
NW = 32
NCH = 40
CH = 128
EP = NW * NCH * CH
PAD_ROW = NT

NCH0 = 64
NCH1 = 16
NCHMX = max(NCH0, NCH1)
EPM_ROWS = 16 * NCH0 + 15 * NCH1 + NCHMX

APW = 80
AP = NW * APW

@functools.cache
def _mesh():
    return plsc.VectorSubcoreMesh(core_axis_name="c", subcore_axis_name="s",
                                  num_cores=2, num_subcores=16)


def _sc_deg_body(dstm_hbm, zeros_hbm, ones_hbm, out_hbm, didx, ones,
                 accum):
    c = lax.axis_index("c")
    s = lax.axis_index("s")
    wid = c * 16 + s
    rps = NTP // 16

    pltpu.sync_copy(zeros_hbm.at[pl.ds(s * rps, rps)],
                    accum.at[pl.ds(s * rps, rps)])
    pltpu.sync_copy(ones_hbm, ones)
    plsc.subcore_barrier()
    pltpu.sync_copy(dstm_hbm.at[pl.ds(wid * NCH, NCH)], didx)

    def _count(j, _):
        pltpu.sync_copy(ones, accum.at[didx.at[j]], add=True)
        return _

    lax.fori_loop(0, NCH, _count, None)
    plsc.subcore_barrier()
    pltpu.sync_copy(accum.at[pl.ds(s * rps, rps)],
                    out_hbm.at[c, pl.ds(s * rps, rps)])


@functools.cache
def _sc_deg():
    return pl.kernel(
        _sc_deg_body,
        out_type=jax.ShapeDtypeStruct((2, NTP, H), jnp.float32),
        mesh=_mesh(),
        scratch_types=[
            pltpu.VMEM((NCH, CH), jnp.int32),
            pltpu.VMEM((CH, H), jnp.float32),
            pltpu.VMEM_SHARED((NTP, H), jnp.float32),
        ],
    )


def _sc_agg_body(h_hbm, srcm_hbm, dstm_hbm, zeros_hbm, out_hbm, sidx, didx,
                 rows0, rows1, accum, g0, g1, s0, s1):
    c = lax.axis_index("c")
    s = lax.axis_index("s")
    wid = c * 16 + s
    rps = NTP // 16

    pltpu.sync_copy(zeros_hbm.at[pl.ds(s * rps, rps)], accum.at[pl.ds(s * rps, rps)])
    plsc.subcore_barrier()

    base = jnp.where(c == 0, s * NCH0, 16 * NCH0 + s * NCH1)
    pltpu.sync_copy(srcm_hbm.at[pl.ds(base, NCHMX)], sidx)
    pltpu.sync_copy(dstm_hbm.at[pl.ds(base, NCHMX)], didx)

    def _run(n):
        pltpu.async_copy(h_hbm.at[sidx.at[0]], rows0, g0)

        def _pair(i, _):
            a = i * 2
            b = a + 1

            @pl.when(i > 0)
            def _():
                pltpu.make_async_copy(rows1, accum.at[didx.at[a]], s1).wait()

            pltpu.async_copy(h_hbm.at[sidx.at[b]], rows1, g1)
            pltpu.make_async_copy(h_hbm.at[sidx.at[a]], rows0, g0).wait()
            pltpu.async_copy(rows0, accum.at[didx.at[a]], s0, add=True)
            pltpu.make_async_copy(h_hbm.at[sidx.at[b]], rows1, g1).wait()
            pltpu.make_async_copy(rows0, accum.at[didx.at[a]], s0).wait()

            @pl.when(i < n // 2 - 1)
            def _():
                pltpu.async_copy(h_hbm.at[sidx.at[a + 2]], rows0, g0)

            pltpu.async_copy(rows1, accum.at[didx.at[b]], s1, add=True)
            return _

        lax.fori_loop(0, n // 2, _pair, None)
        pltpu.make_async_copy(rows1, accum.at[didx.at[0]], s1).wait()

    @pl.when(c == 0)
    def _():
        _run(NCH0)

    @pl.when(c == 1)
    def _():
        _run(NCH1)

    plsc.subcore_barrier()
    pltpu.sync_copy(accum.at[pl.ds(s * rps, rps)],
                    out_hbm.at[c, pl.ds(s * rps, rps)])


@functools.cache
def _sc_agg():
    return pl.kernel(
        _sc_agg_body,
        out_type=jax.ShapeDtypeStruct((2, NTP, H), jnp.float32),
        mesh=_mesh(),
        scratch_types=[
            pltpu.VMEM((NCHMX, CH), jnp.int32),
            pltpu.VMEM((NCHMX, CH), jnp.int32),
            pltpu.VMEM((CH, H), jnp.float32),
            pltpu.VMEM((CH, H), jnp.float32),
            pltpu.VMEM_SHARED((NTP, H), jnp.float32),
            pltpu.SemaphoreType.DMA,
            pltpu.SemaphoreType.DMA,
            pltpu.SemaphoreType.DMA,
            pltpu.SemaphoreType.DMA,
        ],
    )


def _sc_tail_gather_body(z_hbm, gum_hbm, ridm_hbm, cidm_hbm, gidm_hbm,
                         zr_hbm, zc_hbm, gg_hbm, ridx, cidx, gidx, rrows,
                         crows, grows, sem):
    c = lax.axis_index("c")
    s = lax.axis_index("s")
    wid = c * 16 + s
    pltpu.sync_copy(ridm_hbm, ridx)
    pltpu.sync_copy(cidm_hbm, cidx)
    pltpu.sync_copy(gidm_hbm, gidx)
    pltpu.async_copy(z_hbm.at[ridx.at[wid]], rrows, sem).wait()
    pltpu.sync_copy(rrows, zr_hbm.at[pl.ds(wid * APW, APW)])
    pltpu.async_copy(z_hbm.at[cidx.at[wid]], crows, sem).wait()
    pltpu.sync_copy(crows, zc_hbm.at[pl.ds(wid * APW, APW)])
    pltpu.async_copy(gum_hbm.at[gidx.at[wid]], grows, sem).wait()
    pltpu.sync_copy(grows, gg_hbm.at[pl.ds(wid * APW, APW)])


@functools.cache
def _sc_tail_gather():
    return pl.kernel(
        _sc_tail_gather_body,
        out_type=(
            jax.ShapeDtypeStruct((AP, H), jnp.float32),
            jax.ShapeDtypeStruct((AP, H), jnp.float32),
            jax.ShapeDtypeStruct((AP, H), jnp.float32),
        ),
        mesh=_mesh(),
        scratch_types=[
            pltpu.VMEM((NW, APW), jnp.int32),
            pltpu.VMEM((NW, APW), jnp.int32),
            pltpu.VMEM((NW, APW), jnp.int32),
            pltpu.VMEM((APW, H), jnp.float32),
            pltpu.VMEM((APW, H), jnp.float32),
            pltpu.VMEM((APW, H), jnp.float32),
            pltpu.SemaphoreType.DMA,
        ],
    )


def _recip_precise(v):
    r = jax.lax.reciprocal(v)
    r = r * (2.0 - v * r)
    return r * (2.0 - v * r)


def _rsqrt_precise(v):
    r = jax.lax.rsqrt(v)
    r = r * (1.5 - 0.5 * v * r * r)
    return r * (1.5 - 0.5 * v * r * r)


def _prep_body(d_ref, out_ref):
    ssum = d_ref[0] + d_ref[1]
    out_ref[...] = _recip_precise(jnp.maximum(ssum, 1.0))


def _tc_prep(degp):
    return pl.pallas_call(
        _prep_body,
        out_shape=jax.ShapeDtypeStruct((NTP, H), jnp.float32),
    )(degp)


def _layer_body(h_ref, aggp_ref, invd_ref, w1_ref, b1_ref, w2_ref, b2_ref,
                ga_ref, be_ref, out_ref):
    agg = (aggp_ref[0, :NT, :] + aggp_ref[1, :NT, :]) * invd_ref[...]
    x = h_ref[:NT, :] + agg
    z = jnp.dot(x, w1_ref[...], preferred_element_type=jnp.float32) + b1_ref[...]
    z = jnp.maximum(z, 0.0)
    z = jnp.dot(z, w2_ref[...], preferred_element_type=jnp.float32) + b2_ref[...]
    z = jnp.maximum(z, 0.0)
    mu = jnp.mean(z, axis=0, keepdims=True)
    var = jnp.mean((z - mu) ** 2, axis=0, keepdims=True)
    scale = _rsqrt_precise(var + 1e-5) * ga_ref[...]
    out_ref[:NT, :] = (z - mu) * scale + be_ref[...]
    out_ref[NT:, :] = jnp.zeros((NTP - NT, H), jnp.float32)


def _tc_layer(h_pad, aggp, inv_deg, w1, b1, w2, b2, ga, be):
    return pl.pallas_call(
        _layer_body,
        out_shape=jax.ShapeDtypeStruct((NTP, H), jnp.float32),
    )(h_pad, aggp, inv_deg, w1, b1, w2, b2, ga, be)


_BLK = 1000
_GPB = _BLK // N


def _policy_body(h1_ref, h2_ref, h3_ref, h4_ref, w1t_ref, w1b_ref, b1_ref,
                 w2_ref, b2_ref, pw1_ref, pb1_ref, pw2_ref, pb2_ref, out_ref):
    npool = h1_ref[...] + h2_ref[...] + h3_ref[...] + h4_ref[...]
    g_of = jax.lax.broadcasted_iota(jnp.int32, (_GPB, _BLK), 0)
    r_of = jax.lax.broadcasted_iota(jnp.int32, (_GPB, _BLK), 1) // N
    pm = jnp.where(g_of == r_of, 1.0 / N, 0.0)
    gp = jnp.dot(pm, npool, preferred_element_type=jnp.float32)
    q_r = jax.lax.broadcasted_iota(jnp.int32, (_BLK, _GPB), 0) // N
    q_g = jax.lax.broadcasted_iota(jnp.int32, (_BLK, _GPB), 1)
    qm = jnp.where(q_r == q_g, 1.0, 0.0)
    t = (jnp.dot(npool, w1t_ref[...], preferred_element_type=jnp.float32)
         + jnp.dot(qm, jnp.dot(gp, w1b_ref[...],
                               preferred_element_type=jnp.float32),
                   preferred_element_type=jnp.float32)
         + b1_ref[...])
    z = jnp.dot(jnp.tanh(t), w2_ref[...],
                preferred_element_type=jnp.float32) + b2_ref[...]
    for i in range(2):
        t = jnp.dot(z, pw1_ref[i], preferred_element_type=jnp.float32) + pb1_ref[i:i + 1, :]
        z = jnp.dot(jnp.tanh(t), pw2_ref[i],
                    preferred_element_type=jnp.float32) + pb2_ref[i:i + 1, :]
    out_ref[...] = z


def _tc_policy(h1, h2, h3, h4, p0_W1, p0_b1, p0_W2, p0_b2, pW1, pb1, pW2, pb2):
    blk = pl.BlockSpec((_BLK, H), lambda i: (i, 0))
    full = lambda *shape: pl.BlockSpec(shape, lambda i: tuple(0 for _ in shape))
    return pl.pallas_call(
        _policy_body,
        grid=(NT // _BLK,),
        in_specs=[blk, blk, blk, blk,
                  full(H, H), full(H, H), full(1, H),
                  full(H, H), full(1, H),
                  full(2, H, H), full(2, H), full(2, H, H), full(2, H)],
        out_specs=blk,
        out_shape=jax.ShapeDtypeStruct((NT, H), jnp.float32),
    )(h1, h2, h3, h4, p0_W1[:H], p0_W1[H:], p0_b1.reshape(1, H),
      p0_W2, p0_b2.reshape(1, H), pW1, pb1, pW2, pb2)


def _tail_body(zr_ref, zc_ref, gg_ref, sub_ref, pm_ref, out_s_ref, out_lp_ref):
    s = jnp.sum(zr_ref[...] * zc_ref[...], axis=-1)
    k16 = jax.lax.broadcasted_iota(jnp.int32, (B, A, H), 2)
    g = jnp.sum(jnp.where(k16 == sub_ref[...][:, :, None], gg_ref[...], 0.0),
                axis=-1)
    pm = pm_ref[...]
    i_idx = jax.lax.broadcasted_iota(jnp.int32, (B, A, A), 1)
    j_idx = jax.lax.broadcasted_iota(jnp.int32, (B, A, A), 2)
    dup = jnp.any((pm[:, :, None] == pm[:, None, :]) & (i_idx > j_idx), axis=2)
    tot = s + g
    m = jnp.max(tot, axis=1, keepdims=True)
    ia = jax.lax.broadcasted_iota(jnp.int32, (B, A), 1)
    win = jnp.min(jnp.where(tot == m, ia, A), axis=1, keepdims=True)
    sel = ia == win
    aid = jnp.sum(jnp.where(sel, pm, 0), axis=1, keepdims=True)
    s_win = jnp.sum(jnp.where(sel, s, 0.0), axis=1, keepdims=True)
    s_mask = jnp.where(dup, -jnp.inf, s)
    mm = jnp.max(s_mask, axis=1, keepdims=True)
    lse = mm + jnp.log(jnp.sum(jnp.where(dup, 0.0, jnp.exp(s - mm)),
                               axis=1, keepdims=True))
    out_lp_ref[...] = s_win - lse
    out_s_ref[:, 0:1] = aid // N
    out_s_ref[:, 1:2] = aid % N


def _tc_tail(zr, zc, gg, sub, pm):
    return pl.pallas_call(
        _tail_body,
        out_shape=(jax.ShapeDtypeStruct((B, 2), jnp.int32),
                   jax.ShapeDtypeStruct((B, 1), jnp.float32)),
    )(zr, zc, gg, sub, pm)


def kernel(x, edge_index, batch, feasible_actions, gin_W1, gin_b1, gin_W2,
           gin_b2, bn_gamma, bn_beta, p0_W1, p0_b1, p0_W2, p0_b2, pW1, pb1,
           pW2, pb2):
    src = edge_index[0]
    dst = edge_index[1]
    srcm = jnp.full((EPM_ROWS * CH,), PAD_ROW, jnp.int32).at[:E].set(src).reshape(EPM_ROWS, CH)
    dstm = jnp.full((EPM_ROWS * CH,), PAD_ROW, jnp.int32).at[:E].set(dst).reshape(EPM_ROWS, CH)
    zeros_pad = jnp.zeros((NTP, H), jnp.float32)

    degp = _sc_deg()(dstm, zeros_pad, jnp.ones((CH, H), jnp.float32))
    inv_deg = _tc_prep(degp)[:NT, :1]

    h = jnp.zeros((NTP, H), jnp.float32).at[:NT].set(x)
    hs = []
    for l in range(L):
        aggp = _sc_agg()(h, srcm, dstm, zeros_pad)
        h = _tc_layer(h, aggp, inv_deg, gin_W1[l], gin_b1[l].reshape(1, H),
                      gin_W2[l], gin_b2[l].reshape(1, H),
                      bn_gamma[l].reshape(1, H), bn_beta[l].reshape(1, H))
        hs.append(h)

    z = _tc_policy(hs[0], hs[1], hs[2], hs[3], p0_W1, p0_b1, p0_W2, p0_b2,
                   pW1, pb1, pW2, pb2)

    r = feasible_actions[:, :, 0]
    c = feasible_actions[:, :, 1]
    pmat = r * N + c
    boff = jnp.arange(B, dtype=jnp.int32)[:, None]
    ridx = jnp.zeros((AP,), jnp.int32).at[:B * A].set((r + boff * N).ravel())
    cidx = jnp.zeros((AP,), jnp.int32).at[:B * A].set((c + boff * N).ravel())
    gpos = (pmat + boff * (N * N)).ravel()
    gidx = jnp.zeros((AP,), jnp.int32).at[:B * A].set(gpos // H)
    sub = (gpos % H).astype(jnp.int32).reshape(B, A)

    gum = jax.random.gumbel(jax.random.key(42), (B, 1, N * N), jnp.float32)
    gum16 = gum.reshape(B * N * N // H, H)

    zr, zc, gg = _sc_tail_gather()(z, gum16, ridx.reshape(NW, APW),
                                   cidx.reshape(NW, APW), gidx.reshape(NW, APW))
    sampled2, lp = _tc_tail(zr[:B * A].reshape(B, A, H),
                            zc[:B * A].reshape(B, A, H),
                            gg[:B * A].reshape(B, A, H), sub, pmat)
    return (sampled2.reshape(B, 1, 2), lp)
